# Initial kernel scaffold; baseline (speedup 1.0000x reference)
import jax
import jax.numpy as jnp
from jax.experimental import pallas as pl


def kernel(img_feat, graph_feat, p3d_normed, roi_mask_bit, prev_x_id, prev_y_id, knn_idx, params):
    raise NotImplementedError("write your pallas kernel here")


# speedup vs baseline: 4.6045x; 4.6045x over previous
import functools

import jax
import jax.numpy as jnp
from jax import lax
from jax.experimental import pallas as pl
from jax.experimental.pallas import tpu as pltpu
from jax.experimental.pallas import tpu_sc as plsc

B = 4
N = 1024
K = 20
C = 256
GF = 64
EMB = 64
PG = 34
NPOS = PG * PG
NTOK = B * N
NW = 32
TPW = NTOK // NW
TOKBLK = 512
NBLK = NTOK // TOKBLK
POUT = 1192
PFLAT = 1232

_SC_MESH = plsc.VectorSubcoreMesh(core_axis_name="c", subcore_axis_name="s")


def _leaky(x, s):
    return jnp.where(x >= 0, x, s * x)


def _conv_body(p_ref, w_ref, b_ref, o_ref):
    acc = jnp.zeros((128, POUT), jnp.float32)
    for t in range(16):
        i, j = t // 4, t % 4
        q = (i % 2) * 2 + (j % 2)
        off = (i // 2) * 35 + (j // 2)
        rhs = p_ref[0, q, :, off:off + POUT]
        acc = acc + jnp.dot(w_ref[t], rhs, preferred_element_type=jnp.float32)
    o_ref[0] = acc + b_ref[...]


def _conv_matmul(planes, w_conv, conv_b):
    return pl.pallas_call(
        _conv_body,
        grid=(B,),
        in_specs=[
            pl.BlockSpec((1, 4, C, PFLAT), lambda b: (b, 0, 0, 0)),
            pl.BlockSpec((16, 128, C), lambda b: (0, 0, 0)),
            pl.BlockSpec((128, 1), lambda b: (0, 0)),
        ],
        out_specs=pl.BlockSpec((1, 128, POUT), lambda b: (b, 0, 0)),
        out_shape=jax.ShapeDtypeStruct((B, 128, POUT), jnp.float32),
    )(planes, w_conv, conv_b)


@functools.partial(
    pl.kernel,
    mesh=_SC_MESH,
    out_type=jax.ShapeDtypeStruct((NTOK * 4, 128), jnp.float32),
    scratch_types=[
        pltpu.VMEM((4, 128), jnp.int32),
        pltpu.VMEM((128, 128), jnp.float32),
        pltpu.SemaphoreType.DMA,
    ],
)
def _patch_gather(pat_hbm, idx_hbm, out_hbm, idx_v, rows_v, sem):
    wid = lax.axis_index("s") * 2 + lax.axis_index("c")
    base = wid * 512
    pltpu.sync_copy(idx_hbm.at[wid], idx_v)
    for g in range(4):
        pltpu.async_copy(pat_hbm.at[idx_v.at[g]], rows_v, sem).wait()
        pltpu.sync_copy(rows_v, out_hbm.at[pl.ds(base + g * 128, 128)])


@functools.partial(
    pl.kernel,
    mesh=_SC_MESH,
    out_type=(
        jax.ShapeDtypeStruct((NTOK, C), jnp.float32),
        jax.ShapeDtypeStruct((NTOK, C), jnp.float32),
        jax.ShapeDtypeStruct((NTOK, C), jnp.float32),
    ),
    scratch_types=[
        pltpu.VMEM((TPW, K), jnp.int32),
        pltpu.VMEM((K, C), jnp.float32),
        pltpu.VMEM((TPW, C), jnp.float32),
        pltpu.VMEM((TPW, C), jnp.float32),
        pltpu.VMEM((TPW, C), jnp.float32),
        pltpu.SemaphoreType.DMA,
    ],
)
def _knn_reduce(z_hbm, idx_hbm, mx_hbm, mn_hbm, sm_hbm,
                idx_v, rows_v, mx_v, mn_v, sm_v, sem):
    wid = lax.axis_index("s") * 2 + lax.axis_index("c")
    base = wid * TPW
    pltpu.sync_copy(idx_hbm.at[pl.ds(base, TPW)], idx_v)

    def body(p, carry):
        pltpu.async_copy(z_hbm.at[idx_v.at[p]], rows_v, sem).wait()
        for cc in range(C // 16):
            col = pl.ds(cc * 16, 16)
            v = rows_v[0, col]
            mx = v
            mn = v
            sm = v
            for j in range(1, K):
                v = rows_v[j, col]
                mx = jnp.maximum(mx, v)
                mn = jnp.minimum(mn, v)
                sm = sm + v
            mx_v[p, col] = mx
            mn_v[p, col] = mn
            sm_v[p, col] = sm
        return carry

    lax.fori_loop(0, TPW, body, 0)
    pltpu.sync_copy(mx_v, mx_hbm.at[pl.ds(base, TPW)])
    pltpu.sync_copy(mn_v, mn_hbm.at[pl.ds(base, TPW)])
    pltpu.sync_copy(sm_v, sm_hbm.at[pl.ds(base, TPW)])


def _pre_body(loc_ref, gph_ref, roi_ref, cnt_ref, w1a_ref, w1b_ref, b1_ref,
              w2_ref, b2_ref, wz_ref, wu_ref, o_z, o_c, o_st):
    i = pl.program_id(0)
    r = roi_ref[...][:, 0:1]
    h = jnp.dot(loc_ref[...] * r, w1a_ref[...],
                preferred_element_type=jnp.float32)
    h = h + jnp.dot(gph_ref[...], w1b_ref[...],
                    preferred_element_type=jnp.float32)
    h = _leaky(h + b1_ref[...], 0.01)
    x = _leaky(jnp.dot(h, w2_ref[...], preferred_element_type=jnp.float32)
               + b2_ref[...], 0.01)
    z = jnp.dot(x, wz_ref[...], preferred_element_type=jnp.float32)
    u = jnp.dot(x, wu_ref[...], preferred_element_type=jnp.float32)
    c = u - z
    o_z[...] = z
    o_c[...] = c
    cnt = cnt_ref[...][:, 0:1]
    st = jnp.concatenate(
        [
            jnp.sum(cnt * z, axis=0, keepdims=True),
            jnp.sum(cnt * z * z, axis=0, keepdims=True),
            jnp.sum(c, axis=0, keepdims=True),
            jnp.sum(c * c, axis=0, keepdims=True),
        ],
        axis=0,
    )

    @pl.when(i == 0)
    def _():
        o_st[...] = st

    @pl.when(i > 0)
    def _():
        o_st[...] = o_st[...] + st


def _pre_mlp(local, graph_tok, roi_tok, cnt_tok, w1a, w1b, b1, w2, b2, wz, wu):
    return pl.pallas_call(
        _pre_body,
        grid=(NBLK,),
        in_specs=[
            pl.BlockSpec((TOKBLK, 2 * C), lambda i: (i, 0)),
            pl.BlockSpec((TOKBLK, GF), lambda i: (i, 0)),
            pl.BlockSpec((TOKBLK, 8), lambda i: (i, 0)),
            pl.BlockSpec((TOKBLK, 8), lambda i: (i, 0)),
            pl.BlockSpec((2 * C, C), lambda i: (0, 0)),
            pl.BlockSpec((GF, C), lambda i: (0, 0)),
            pl.BlockSpec((1, C), lambda i: (0, 0)),
            pl.BlockSpec((C, C), lambda i: (0, 0)),
            pl.BlockSpec((1, C), lambda i: (0, 0)),
            pl.BlockSpec((C, C), lambda i: (0, 0)),
            pl.BlockSpec((C, C), lambda i: (0, 0)),
        ],
        out_specs=[
            pl.BlockSpec((TOKBLK, C), lambda i: (i, 0)),
            pl.BlockSpec((TOKBLK, C), lambda i: (i, 0)),
            pl.BlockSpec((4, C), lambda i: (0, 0)),
        ],
        out_shape=[
            jax.ShapeDtypeStruct((NTOK, C), jnp.float32),
            jax.ShapeDtypeStruct((NTOK, C), jnp.float32),
            jax.ShapeDtypeStruct((4, C), jnp.float32),
        ],
    )(local, graph_tok, roi_tok, cnt_tok, w1a, w1b, b1, w2, b2, wz, wu)


def _stats_body(c_ref, s_ref, st_ref, g_ref, b_ref, o_ref, acc_ref):
    i = pl.program_id(0)

    @pl.when(i == 0)
    def _():
        acc_ref[...] = jnp.zeros_like(acc_ref)

    acc_ref[...] = acc_ref[...] + jnp.sum(c_ref[...] * s_ref[...], axis=0,
                                          keepdims=True)

    @pl.when(i == pl.num_programs(0) - 1)
    def _():
        st = st_ref[...]
        denom = float(B * N * K)
        mean = (st[0:1] + K * st[2:3]) / denom
        ey2 = (st[1:2] + 2.0 * acc_ref[...] + K * st[3:4]) / denom
        var = ey2 - mean * mean
        a = g_ref[...] * lax.rsqrt(var + 1e-5)
        bias = b_ref[...] - mean * a
        o_ref[...] = jnp.concatenate([a, bias], axis=0)


def _bn_stats(cs, ss, st, gamma, beta):
    return pl.pallas_call(
        _stats_body,
        grid=(NBLK,),
        in_specs=[
            pl.BlockSpec((TOKBLK, C), lambda i: (i, 0)),
            pl.BlockSpec((TOKBLK, C), lambda i: (i, 0)),
            pl.BlockSpec((4, C), lambda i: (0, 0)),
            pl.BlockSpec((1, C), lambda i: (0, 0)),
            pl.BlockSpec((1, C), lambda i: (0, 0)),
        ],
        out_specs=pl.BlockSpec((2, C), lambda i: (0, 0)),
        out_shape=jax.ShapeDtypeStruct((2, C), jnp.float32),
        scratch_shapes=[pltpu.VMEM((1, C), jnp.float32)],
    )(cs, ss, st, gamma, beta)


def _apply0_body(c_ref, mx_ref, mn_ref, ab_ref, cnt_ref, wz_ref, wu_ref,
                 o_z, o_c, o_st):
    i = pl.program_id(0)
    a = ab_ref[...][0:1]
    bias = ab_ref[...][1:2]
    sel = jnp.where(a >= 0, mx_ref[...], mn_ref[...])
    y = a * (sel + c_ref[...]) + bias
    x = _leaky(y, 0.2)
    z = jnp.dot(x, wz_ref[...], preferred_element_type=jnp.float32)
    u = jnp.dot(x, wu_ref[...], preferred_element_type=jnp.float32)
    c = u - z
    o_z[...] = z
    o_c[...] = c
    cnt = cnt_ref[...][:, 0:1]
    st = jnp.concatenate(
        [
            jnp.sum(cnt * z, axis=0, keepdims=True),
            jnp.sum(cnt * z * z, axis=0, keepdims=True),
            jnp.sum(c, axis=0, keepdims=True),
            jnp.sum(c * c, axis=0, keepdims=True),
        ],
        axis=0,
    )

    @pl.when(i == 0)
    def _():
        o_st[...] = st

    @pl.when(i > 0)
    def _():
        o_st[...] = o_st[...] + st


def _apply0(c1, mx1, mn1, ab1, cnt_tok, wz, wu):
    return pl.pallas_call(
        _apply0_body,
        grid=(NBLK,),
        in_specs=[
            pl.BlockSpec((TOKBLK, C), lambda i: (i, 0)),
            pl.BlockSpec((TOKBLK, C), lambda i: (i, 0)),
            pl.BlockSpec((TOKBLK, C), lambda i: (i, 0)),
            pl.BlockSpec((2, C), lambda i: (0, 0)),
            pl.BlockSpec((TOKBLK, 8), lambda i: (i, 0)),
            pl.BlockSpec((C, C), lambda i: (0, 0)),
            pl.BlockSpec((C, C), lambda i: (0, 0)),
        ],
        out_specs=[
            pl.BlockSpec((TOKBLK, C), lambda i: (i, 0)),
            pl.BlockSpec((TOKBLK, C), lambda i: (i, 0)),
            pl.BlockSpec((4, C), lambda i: (0, 0)),
        ],
        out_shape=[
            jax.ShapeDtypeStruct((NTOK, C), jnp.float32),
            jax.ShapeDtypeStruct((NTOK, C), jnp.float32),
            jax.ShapeDtypeStruct((4, C), jnp.float32),
        ],
    )(c1, mx1, mn1, ab1, cnt_tok, wz, wu)


def _apply1_body(c_ref, mx_ref, mn_ref, ab_ref, qw1_ref, qb1_ref, qw2_ref,
                 qb2_ref, qw3_ref, qb3_ref, o_x, o_q):
    a = ab_ref[...][0:1]
    bias = ab_ref[...][1:2]
    sel = jnp.where(a >= 0, mx_ref[...], mn_ref[...])
    x = _leaky(a * (sel + c_ref[...]) + bias, 0.2)
    q = _leaky(jnp.dot(x, qw1_ref[...], preferred_element_type=jnp.float32)
               + qb1_ref[...], 0.01)
    q = _leaky(jnp.dot(q, qw2_ref[...], preferred_element_type=jnp.float32)
               + qb2_ref[...], 0.01)
    q = jnp.dot(q, qw3_ref[...], preferred_element_type=jnp.float32) \
        + qb3_ref[...]
    o_x[...] = x
    o_q[...] = q


def _apply1(c2, mx2, mn2, ab2, qw1, qb1, qw2, qb2, qw3p, qb3p):
    return pl.pallas_call(
        _apply1_body,
        grid=(NBLK,),
        in_specs=[
            pl.BlockSpec((TOKBLK, C), lambda i: (i, 0)),
            pl.BlockSpec((TOKBLK, C), lambda i: (i, 0)),
            pl.BlockSpec((TOKBLK, C), lambda i: (i, 0)),
            pl.BlockSpec((2, C), lambda i: (0, 0)),
            pl.BlockSpec((C, C), lambda i: (0, 0)),
            pl.BlockSpec((1, C), lambda i: (0, 0)),
            pl.BlockSpec((C, GF), lambda i: (0, 0)),
            pl.BlockSpec((1, GF), lambda i: (0, 0)),
            pl.BlockSpec((GF, 128), lambda i: (0, 0)),
            pl.BlockSpec((1, 128), lambda i: (0, 0)),
        ],
        out_specs=[
            pl.BlockSpec((TOKBLK, C), lambda i: (i, 0)),
            pl.BlockSpec((TOKBLK, 128), lambda i: (i, 0)),
        ],
        out_shape=[
            jax.ShapeDtypeStruct((NTOK, C), jnp.float32),
            jax.ShapeDtypeStruct((NTOK, 128), jnp.float32),
        ],
    )(c2, mx2, mn2, ab2, qw1, qb1, qw2, qb2, qw3p, qb3p)


def kernel(img_feat, graph_feat, p3d_normed, roi_mask_bit, prev_x_id,
           prev_y_id, knn_idx, params):
    del p3d_normed
    p = params

    imgp = jnp.pad(img_feat, ((0, 0), (0, 0), (3, 3), (3, 3)))
    planes = imgp.reshape(B, C, 35, 2, 35, 2)
    planes = planes.transpose(0, 3, 5, 1, 2, 4).reshape(B, 4, C, 1225)
    planes = jnp.pad(planes, ((0, 0), (0, 0), (0, 0), (0, PFLAT - 1225)))
    w_conv = p['conv_w'].transpose(2, 3, 0, 1).reshape(16, EMB, C)
    w_conv = jnp.pad(w_conv, ((0, 0), (0, 128 - EMB), (0, 0)))
    conv_b = jnp.pad(p['conv_b'], (0, 128 - EMB))[:, None]
    patches_cm = _conv_matmul(planes, w_conv, conv_b)
    patches = patches_cm.transpose(0, 2, 1).reshape(B * POUT, 128)

    y_id = prev_y_id.astype(jnp.int32)
    x_id = prev_x_id.astype(jnp.int32)
    boff = (jnp.arange(B, dtype=jnp.int32) * POUT)[:, None, None]
    pos = jnp.stack(
        [y_id * 35 + x_id, (y_id + 2) * 35 + x_id,
         y_id * 35 + x_id + 2, (y_id + 2) * 35 + x_id + 2], axis=-1)
    pidx = (boff + pos).reshape(NW, 4, 128)
    local = _patch_gather(patches, pidx).reshape(NTOK, 2 * C)

    graph_tok = graph_feat.transpose(0, 2, 1).reshape(NTOK, GF)
    roi_tok = jnp.broadcast_to(
        roi_mask_bit.transpose(0, 2, 1).reshape(NTOK, 1), (NTOK, 8))

    idx = knn_idx[0].astype(jnp.int32)
    gidx = (jnp.arange(B, dtype=jnp.int32)[:, None, None] * N
            + idx[None]).reshape(NTOK, K)
    count = jnp.zeros((N,), jnp.float32).at[idx.reshape(-1)].add(1.0)
    cnt_tok = jnp.broadcast_to(
        jnp.tile(count, B).reshape(NTOK, 1), (NTOK, 8))

    w1a_pad = jnp.pad(p['pre_w1'][:C].reshape(4, EMB, C),
                      ((0, 0), (0, 128 - EMB), (0, 0))).reshape(2 * C, C)
    w0 = p['gconv_w0']
    w1 = p['gconv_w1']
    z1, c1, st0 = _pre_mlp(
        local, graph_tok, roi_tok, cnt_tok,
        w1a_pad, p['pre_w1'][C:], p['pre_b1'][None, :],
        p['pre_w2'], p['pre_b2'][None, :],
        w0[:, :C].T, w0[:, C:].T)

    mx1, mn1, sm1 = _knn_reduce(z1, gidx)
    ab1 = _bn_stats(c1, sm1, st0, p['gamma0'][None, :], p['beta0'][None, :])
    z2, c2, st1 = _apply0(c1, mx1, mn1, ab1, cnt_tok,
                          w1[:, :C].T, w1[:, C:].T)

    mx2, mn2, sm2 = _knn_reduce(z2, gidx)
    ab2 = _bn_stats(c2, sm2, st1, p['gamma1'][None, :], p['beta1'][None, :])

    qw3p = jnp.pad(p['qw3'], ((0, 0), (0, 126)))
    qb3p = jnp.pad(p['qb3'], (0, 126))[None, :]
    x3, q3 = _apply1(c2, mx2, mn2, ab2,
                     p['qw1'], p['qb1'][None, :],
                     p['qw2'], p['qb2'][None, :], qw3p, qb3p)

    output_bits = q3[:, :2].reshape(B, N, 2).transpose(0, 2, 1)
    x_out = x3.reshape(B, N, C).transpose(0, 2, 1)
    return output_bits, x_out
